# baseline (speedup 1.0000x reference)
import functools

import jax
import jax.numpy as jnp
from jax import lax
from jax.experimental import pallas as pl
from jax.experimental.pallas import tpu as pltpu
from jax.experimental.pallas import tpu_sc as plsc

VOCAB = 9
NSEG = 2
NROWS = VOCAB * NSEG
NPAIR = NROWS * NROWS
D = 1024
D2 = 2 * D
NREP = 4


def _prep_kernel(wrows, xe_ref, xo_ref, se_ref, so_ref, tok_ref, segt_ref,
                 gamma_ref, beta_ref, idx_ref, p_ref):
    t = tok_ref[...]
    s0 = segt_ref[0:1, :]
    s1 = segt_ref[1:2, :]
    e = jnp.concatenate([t + s0, t + s1], axis=0)
    mean = jnp.mean(e, axis=-1, keepdims=True)
    ctr = e - mean
    var = jnp.mean(ctr * ctr, axis=-1, keepdims=True)
    normed = ctr * lax.rsqrt(var + 1e-5)
    f = normed * gamma_ref[...] + beta_ref[...]
    left = jnp.broadcast_to(f[:, None, :], (NROWS, NROWS, D)).reshape(
        NPAIR, D)
    right = jnp.broadcast_to(f[None, :, :], (NROWS, NROWS, D)).reshape(
        NPAIR, D)
    p = jnp.concatenate([left, right], axis=1)
    p_ref[...] = jnp.broadcast_to(p[None], (NREP, NPAIR, D2)).reshape(
        NREP * NPAIR, D2)
    rep = lax.broadcasted_iota(jnp.int32, xe_ref.shape, 0) // (
        wrows * (32 // NREP))
    ia = xe_ref[...] + VOCAB * se_ref[...]
    ib = xo_ref[...] + VOCAB * so_ref[...]
    idx_ref[...] = ia * NROWS + ib + NPAIR * rep


def _make_sc_gather(n_pairs):
    info = plsc.get_sparse_core_info()
    nc, ns = info.num_cores, info.num_subcores
    nw = nc * ns
    per_w = n_pairs // nw
    chunk = 16
    n_chunks = per_w // chunk

    mesh = plsc.VectorSubcoreMesh(core_axis_name="c", subcore_axis_name="s")

    @functools.partial(
        pl.kernel,
        mesh=mesh,
        out_type=jax.ShapeDtypeStruct((n_pairs, D2), jnp.float32),
        scratch_types=[
            pltpu.VMEM((per_w,), jnp.int32),
            pltpu.VMEM((chunk, D2), jnp.float32),
            pltpu.VMEM((chunk, D2), jnp.float32),
            pltpu.VMEM((chunk, D2), jnp.float32),
            pltpu.SemaphoreType.DMA,
            pltpu.SemaphoreType.DMA,
            pltpu.SemaphoreType.DMA,
            pltpu.SemaphoreType.DMA,
            pltpu.SemaphoreType.DMA,
            pltpu.SemaphoreType.DMA,
        ],
    )
    def sc_gather(p_hbm, idx_hbm, out_hbm, idx_v,
                  buf0, buf1, buf2, gs0, gs1, gs2, ws0, ws1, ws2):
        wid = lax.axis_index("s") * nc + lax.axis_index("c")
        base = wid * per_w
        pltpu.sync_copy(idx_hbm.at[pl.ds(base, per_w)], idx_v)
        bufs = (buf0, buf1, buf2)
        gsems = (gs0, gs1, gs2)
        wsems = (ws0, ws1, ws2)

        def gather(c):
            return pltpu.async_copy(
                p_hbm.at[idx_v.at[pl.ds(c * chunk, chunk)]],
                bufs[c % 3], gsems[c % 3])

        def write(c):
            return pltpu.async_copy(
                bufs[c % 3], out_hbm.at[pl.ds(base + c * chunk, chunk)],
                wsems[c % 3])

        gcopies = [None, None, None]
        wcopies = [None, None, None]
        gcopies[0] = gather(0)
        gcopies[1] = gather(1)
        for c in range(n_chunks):
            nxt = c + 2
            if nxt < n_chunks:
                if c >= 1:
                    wcopies[nxt % 3].wait()
                gcopies[nxt % 3] = gather(nxt)
            gcopies[c % 3].wait()
            wcopies[c % 3] = write(c)
        for c in range(max(0, n_chunks - 3), n_chunks):
            wcopies[c % 3].wait()

    return sc_gather, nw, per_w


def kernel(x, seg, tok_table, seg_table, gamma, beta):
    B, L = x.shape
    n_tokens = B * L
    n_pairs = n_tokens // 2
    sc_gather, nw, per_w = _make_sc_gather(n_pairs)
    wrows = per_w // 128

    xp = x.reshape(n_pairs, 2)
    sp = seg.reshape(n_pairs, 2)
    shape2d = (n_pairs // 128, 128)

    idx2d, ptab = pl.pallas_call(
        functools.partial(_prep_kernel, wrows),
        out_shape=(
            jax.ShapeDtypeStruct(shape2d, jnp.int32),
            jax.ShapeDtypeStruct((NREP * NPAIR, D2), jnp.float32),
        ),
    )(
        xp[:, 0].reshape(shape2d),
        xp[:, 1].reshape(shape2d),
        sp[:, 0].reshape(shape2d),
        sp[:, 1].reshape(shape2d),
        tok_table,
        seg_table,
        gamma.reshape(1, D),
        beta.reshape(1, D),
    )

    idx = idx2d.reshape(n_pairs)
    out = sc_gather(ptab, idx)
    return out.reshape(B, L, D)


# speedup vs baseline: 2.2968x; 2.2968x over previous
import functools

import jax
import jax.numpy as jnp
from jax import lax
from jax.experimental import pallas as pl
from jax.experimental.pallas import tpu as pltpu
from jax.experimental.pallas import tpu_sc as plsc

VOCAB = 9
NSEG = 2
NROWS = VOCAB * NSEG
D = 1024


def _prep_kernel(nworkers, wdiv, x_ref, seg_ref, tok_ref, segt_ref,
                 gamma_ref, beta_ref, idx_ref, f_ref):
    t = tok_ref[...]
    s0 = segt_ref[0:1, :]
    s1 = segt_ref[1:2, :]
    e = jnp.concatenate([t + s0, t + s1], axis=0)
    mean = jnp.mean(e, axis=-1, keepdims=True)
    ctr = e - mean
    var = jnp.mean(ctr * ctr, axis=-1, keepdims=True)
    normed = ctr * lax.rsqrt(var + 1e-5)
    f = normed * gamma_ref[...] + beta_ref[...]
    f_ref[...] = jnp.broadcast_to(f[None], (nworkers, NROWS, D)).reshape(
        nworkers * NROWS, D)
    w = lax.broadcasted_iota(jnp.int32, x_ref.shape, 0) // wdiv
    idx_ref[...] = x_ref[...] + VOCAB * seg_ref[...] + NROWS * w


def _make_sc_gather(n_tokens):
    info = plsc.get_sparse_core_info()
    nc, ns = info.num_cores, info.num_subcores
    nw = nc * ns
    per_w = n_tokens // nw
    chunk = 32
    n_chunks = per_w // chunk

    mesh = plsc.VectorSubcoreMesh(core_axis_name="c", subcore_axis_name="s")

    @functools.partial(
        pl.kernel,
        mesh=mesh,
        compiler_params=pltpu.CompilerParams(skip_device_barrier=True),
        out_type=jax.ShapeDtypeStruct((n_tokens, D), jnp.float32),
        scratch_types=[
            pltpu.VMEM((per_w,), jnp.int32),
            pltpu.VMEM((chunk, D), jnp.float32),
            pltpu.VMEM((chunk, D), jnp.float32),
            pltpu.VMEM((chunk, D), jnp.float32),
            pltpu.SemaphoreType.DMA,
            pltpu.SemaphoreType.DMA,
            pltpu.SemaphoreType.DMA,
            pltpu.SemaphoreType.DMA,
            pltpu.SemaphoreType.DMA,
            pltpu.SemaphoreType.DMA,
        ],
    )
    def sc_gather(f_hbm, idx_hbm, out_hbm, idx_v,
                  buf0, buf1, buf2, gs0, gs1, gs2, ws0, ws1, ws2):
        wid = lax.axis_index("s") * nc + lax.axis_index("c")
        base = wid * per_w
        pltpu.sync_copy(idx_hbm.at[pl.ds(base, per_w)], idx_v)
        bufs = (buf0, buf1, buf2)
        gsems = (gs0, gs1, gs2)
        wsems = (ws0, ws1, ws2)

        def gather(c):
            return pltpu.async_copy(
                f_hbm.at[idx_v.at[pl.ds(c * chunk, chunk)]],
                bufs[c % 3], gsems[c % 3])

        def write(c):
            return pltpu.async_copy(
                bufs[c % 3], out_hbm.at[pl.ds(base + c * chunk, chunk)],
                wsems[c % 3])

        gcopies = [None, None, None]
        wcopies = [None, None, None]
        gcopies[0] = gather(0)
        gcopies[1] = gather(1)
        for c in range(n_chunks):
            nxt = c + 2
            if nxt < n_chunks:
                if c >= 1:
                    wcopies[nxt % 3].wait()
                gcopies[nxt % 3] = gather(nxt)
            gcopies[c % 3].wait()
            wcopies[c % 3] = write(c)
        for c in range(max(0, n_chunks - 3), n_chunks):
            wcopies[c % 3].wait()

    return sc_gather, nw, per_w


def kernel(x, seg, tok_table, seg_table, gamma, beta):
    B, L = x.shape
    n_tokens = B * L
    sc_gather, nw, per_w = _make_sc_gather(n_tokens)
    wdiv = per_w // 128

    idx2d, ftab = pl.pallas_call(
        functools.partial(_prep_kernel, nw, wdiv),
        out_shape=(
            jax.ShapeDtypeStruct((n_tokens // 128, 128), jnp.int32),
            jax.ShapeDtypeStruct((nw * NROWS, D), jnp.float32),
        ),
    )(
        x.reshape(n_tokens // 128, 128),
        seg.reshape(n_tokens // 128, 128),
        tok_table,
        seg_table,
        gamma.reshape(1, D),
        beta.reshape(1, D),
    )

    idx = idx2d.reshape(n_tokens)
    out = sc_gather(ftab, idx)
    return out.reshape(B, L, D)


# speedup vs baseline: 2.3071x; 1.0045x over previous
import functools

import jax
import jax.numpy as jnp
from jax import lax
from jax.experimental import pallas as pl
from jax.experimental.pallas import tpu as pltpu
from jax.experimental.pallas import tpu_sc as plsc

VOCAB = 9
NSEG = 2
NROWS = VOCAB * NSEG
D = 1024
CHUNK = 56


def _prep_kernel(nworkers, wdiv, x_ref, seg_ref, tok_ref, segt_ref,
                 gamma_ref, beta_ref, idx_ref, f_ref):
    t = tok_ref[...]
    s0 = segt_ref[0:1, :]
    s1 = segt_ref[1:2, :]
    e = jnp.concatenate([t + s0, t + s1], axis=0)
    mean = jnp.mean(e, axis=-1, keepdims=True)
    ctr = e - mean
    var = jnp.mean(ctr * ctr, axis=-1, keepdims=True)
    normed = ctr * lax.rsqrt(var + 1e-5)
    f = normed * gamma_ref[...] + beta_ref[...]
    f_ref[...] = jnp.broadcast_to(f[None], (nworkers, NROWS, D)).reshape(
        nworkers * NROWS, D)
    w = lax.broadcasted_iota(jnp.int32, x_ref.shape, 0) // wdiv
    idx_ref[...] = x_ref[...] + VOCAB * seg_ref[...] + NROWS * w


def _make_sc_gather(n_tokens):
    info = plsc.get_sparse_core_info()
    nc, ns = info.num_cores, info.num_subcores
    nw = nc * ns
    per_w = n_tokens // nw
    sizes = []
    rem = per_w
    while rem >= CHUNK:
        sizes.append(CHUNK)
        rem -= CHUNK
    if rem:
        sizes.append(rem)
    offs = [sum(sizes[:i]) for i in range(len(sizes))]
    n_chunks = len(sizes)

    mesh = plsc.VectorSubcoreMesh(core_axis_name="c", subcore_axis_name="s")

    @functools.partial(
        pl.kernel,
        mesh=mesh,
        out_type=jax.ShapeDtypeStruct((n_tokens, D), jnp.float32),
        scratch_types=[
            pltpu.VMEM((per_w,), jnp.int32),
            pltpu.VMEM((CHUNK, D), jnp.float32),
            pltpu.VMEM((CHUNK, D), jnp.float32),
            pltpu.SemaphoreType.DMA,
            pltpu.SemaphoreType.DMA,
            pltpu.SemaphoreType.DMA,
            pltpu.SemaphoreType.DMA,
        ],
    )
    def sc_gather(f_hbm, idx_hbm, out_hbm, idx_v,
                  buf0, buf1, gs0, gs1, ws0, ws1):
        wid = lax.axis_index("s") * nc + lax.axis_index("c")
        base = wid * per_w
        pltpu.sync_copy(idx_hbm.at[pl.ds(base, per_w)], idx_v)
        bufs = (buf0, buf1)
        gsems = (gs0, gs1)
        wsems = (ws0, ws1)

        def gather(c):
            return pltpu.async_copy(
                f_hbm.at[idx_v.at[pl.ds(offs[c], sizes[c])]],
                bufs[c % 2].at[pl.ds(0, sizes[c])], gsems[c % 2])

        def write(c):
            return pltpu.async_copy(
                bufs[c % 2].at[pl.ds(0, sizes[c])],
                out_hbm.at[pl.ds(base + offs[c], sizes[c])],
                wsems[c % 2])

        gcopies = [None, None]
        wcopies = [None, None]
        gcopies[0] = gather(0)
        for c in range(n_chunks):
            nxt = c + 1
            if nxt < n_chunks:
                if c >= 1:
                    wcopies[nxt % 2].wait()
                gcopies[nxt % 2] = gather(nxt)
            gcopies[c % 2].wait()
            wcopies[c % 2] = write(c)
        for c in range(max(0, n_chunks - 2), n_chunks):
            wcopies[c % 2].wait()

    return sc_gather, nw, per_w


def kernel(x, seg, tok_table, seg_table, gamma, beta):
    B, L = x.shape
    n_tokens = B * L
    sc_gather, nw, per_w = _make_sc_gather(n_tokens)
    wdiv = per_w // 128

    idx2d, ftab = pl.pallas_call(
        functools.partial(_prep_kernel, nw, wdiv),
        out_shape=(
            jax.ShapeDtypeStruct((n_tokens // 128, 128), jnp.int32),
            jax.ShapeDtypeStruct((nw * NROWS, D), jnp.float32),
        ),
    )(
        x.reshape(n_tokens // 128, 128),
        seg.reshape(n_tokens // 128, 128),
        tok_table,
        seg_table,
        gamma.reshape(1, D),
        beta.reshape(1, D),
    )

    idx = idx2d.reshape(n_tokens)
    out = sc_gather(ftab, idx)
    return out.reshape(B, L, D)
